# baseline (speedup 1.0000x reference)
import functools

import jax
import jax.numpy as jnp
from jax import lax
from jax.experimental import pallas as pl
from jax.experimental.pallas import tpu as pltpu
from jax.experimental.pallas import tpu_sc as plsc

N_NODES = 10000
D = 128
NUM_CLASSES = 10
NUM_GRAPHS = 64
NEG_SLOPE = 0.2
E = 160000

_BM = 1000
_GRID = N_NODES // _BM

_INTERPRET = False


def _proj_att_body(x_ref, w_ref, b_ref, att_ref, h_ref, a_ref):
    h = jnp.dot(x_ref[...], w_ref[...], preferred_element_type=jnp.float32)
    h = h + b_ref[...]
    h_ref[...] = h
    a_ref[...] = jnp.dot(h, att_ref[...], preferred_element_type=jnp.float32)


def _proj_att(x, W, b, attcat):
    return pl.pallas_call(
        _proj_att_body,
        grid=(_GRID,),
        in_specs=[
            pl.BlockSpec((_BM, D), lambda i: (i, 0)),
            pl.BlockSpec((D, D), lambda i: (0, 0)),
            pl.BlockSpec((1, D), lambda i: (0, 0)),
            pl.BlockSpec((D, 8), lambda i: (0, 0)),
        ],
        out_specs=[
            pl.BlockSpec((_BM, D), lambda i: (i, 0)),
            pl.BlockSpec((_BM, 8), lambda i: (i, 0)),
        ],
        out_shape=[
            jax.ShapeDtypeStruct((N_NODES, D), jnp.float32),
            jax.ShapeDtypeStruct((N_NODES, 8), jnp.float32),
        ],
        interpret=_INTERPRET,
    )(x, W, b, attcat)


def _combine_a_body(p_ref, kw_ref, kb_ref, o_ref, ssum_ref):
    i = pl.program_id(0)
    o0 = jnp.maximum(p_ref[0, :, :], 0.0)
    o1 = jnp.maximum(p_ref[1, :, :], 0.0)
    o_ref[0, :, :] = o0
    o_ref[1, :, :] = o1
    k0 = jnp.tanh(jnp.dot(o0, kw_ref[...], preferred_element_type=jnp.float32)
                  + kb_ref[...])
    k1 = jnp.tanh(jnp.dot(o1, kw_ref[...], preferred_element_type=jnp.float32)
                  + kb_ref[...])
    s = jnp.stack([jnp.sum(k0, axis=0), jnp.sum(k1, axis=0)], axis=0)

    @pl.when(i == 0)
    def _():
        ssum_ref[...] = s

    @pl.when(i > 0)
    def _():
        ssum_ref[...] = ssum_ref[...] + s


def _combine_a(p, kW, kb):
    return pl.pallas_call(
        _combine_a_body,
        grid=(_GRID,),
        in_specs=[
            pl.BlockSpec((2, _BM, D), lambda i: (0, i, 0)),
            pl.BlockSpec((D, D), lambda i: (0, 0)),
            pl.BlockSpec((1, D), lambda i: (0, 0)),
        ],
        out_specs=[
            pl.BlockSpec((2, _BM, D), lambda i: (0, i, 0)),
            pl.BlockSpec((2, D), lambda i: (0, 0)),
        ],
        out_shape=[
            jax.ShapeDtypeStruct((2, N_NODES, D), jnp.float32),
            jax.ShapeDtypeStruct((2, D), jnp.float32),
        ],
        interpret=_INTERPRET,
    )(p, kW, kb)


def _combine_b_body(apply_relu, o_ref, ssum_ref, q_ref, h_ref):
    t = q_ref[...] * ssum_ref[...] * (1.0 / N_NODES)
    score = jnp.sum(t, axis=1, keepdims=True)
    m = jnp.max(score, axis=0, keepdims=True)
    e = jnp.exp(score - m)
    a = e / jnp.sum(e, axis=0, keepdims=True)
    out = o_ref[0, :, :] * a[0:1, 0:1] + o_ref[1, :, :] * a[1:2, 0:1]
    if apply_relu:
        out = jnp.maximum(out, 0.0)
    h_ref[...] = out


def _combine_b(o, ssum, q, apply_relu):
    return pl.pallas_call(
        functools.partial(_combine_b_body, apply_relu),
        grid=(_GRID,),
        in_specs=[
            pl.BlockSpec((2, _BM, D), lambda i: (0, i, 0)),
            pl.BlockSpec((2, D), lambda i: (0, 0)),
            pl.BlockSpec((1, D), lambda i: (0, 0)),
        ],
        out_specs=pl.BlockSpec((_BM, D), lambda i: (i, 0)),
        out_shape=jax.ShapeDtypeStruct((N_NODES, D), jnp.float32),
        interpret=_INTERPRET,
    )(o, ssum, q)


def _pool_body(h_ref, b_ref, lw_ref, lb_ref, out_ref, psum_ref, pcnt_ref):
    i = pl.program_id(0)
    br = b_ref[0]
    rows = lax.broadcasted_iota(jnp.int32, (NUM_GRAPHS, _BM), 0)
    PT = (rows == br).astype(jnp.float32)
    ps = lax.dot_general(PT, h_ref[...], (((1,), (0,)), ((), ())),
                         preferred_element_type=jnp.float32)
    ones = jnp.ones((_BM, 8), jnp.float32)
    pc = lax.dot_general(PT, ones, (((1,), (0,)), ((), ())),
                         preferred_element_type=jnp.float32)

    @pl.when(i == 0)
    def _():
        psum_ref[...] = ps
        pcnt_ref[...] = pc

    @pl.when(i > 0)
    def _():
        psum_ref[...] = psum_ref[...] + ps
        pcnt_ref[...] = pcnt_ref[...] + pc

    @pl.when(i == _GRID - 1)
    def _():
        pooled = psum_ref[...] / jnp.maximum(pcnt_ref[:, 0:1], 1.0)
        out_ref[...] = jnp.dot(pooled, lw_ref[...],
                               preferred_element_type=jnp.float32) + lb_ref[...]


def _pool_head(h, batch3, lin_W, lin_b):
    return pl.pallas_call(
        _pool_body,
        grid=(_GRID,),
        in_specs=[
            pl.BlockSpec((_BM, D), lambda i: (i, 0)),
            pl.BlockSpec((1, 1, _BM), lambda i: (i, 0, 0)),
            pl.BlockSpec((D, NUM_CLASSES), lambda i: (0, 0)),
            pl.BlockSpec((1, NUM_CLASSES), lambda i: (0, 0)),
        ],
        out_specs=pl.BlockSpec((NUM_GRAPHS, NUM_CLASSES), lambda i: (0, 0)),
        out_shape=jax.ShapeDtypeStruct((NUM_GRAPHS, NUM_CLASSES), jnp.float32),
        scratch_shapes=[
            pltpu.VMEM((NUM_GRAPHS, D), jnp.float32),
            pltpu.VMEM((NUM_GRAPHS, 8), jnp.float32),
        ],
        interpret=_INTERPRET,
    )(h, batch3, lin_W, lin_b)


E_PAD = 163840
N_PAD = 10240
_CHUNKS = 80

_SC_MESH = plsc.VectorSubcoreMesh(core_axis_name="c", subcore_axis_name="s",
                                  num_cores=2, num_subcores=16)


def _edge_a_body(as_hbm, ad_hbm, src_hbm, dst_hbm, ex_hbm, den_hbm,
                 asv, adv, srcv, dstv, exv, zbuf, den_sh):
    c = lax.axis_index("c")
    s = lax.axis_index("s")
    zero16 = jnp.zeros((16,), jnp.float32)
    iota16 = lax.iota(jnp.int32, 16)

    pltpu.sync_copy(as_hbm.at[c], asv)
    pltpu.sync_copy(ad_hbm.at[c], adv)
    pltpu.sync_copy(src_hbm.at[c, s], srcv)
    pltpu.sync_copy(dst_hbm.at[c, s], dstv)

    for t in range(640 // 16):
        zbuf[pl.ds(t * 16, 16)] = zero16
    pltpu.sync_copy(zbuf, den_sh.at[pl.ds(s * 640, 640)])
    plsc.subcore_barrier()

    def _pa(j, _):
        for t in range(8):
            sv = srcv[j, pl.ds(t * 16, 16)]
            dv = dstv[j, pl.ds(t * 16, 16)]
            a = plsc.load_gather(asv, [sv]) + plsc.load_gather(adv, [dv])
            a = jnp.where(a > 0, a, NEG_SLOPE * a)
            ex = jnp.exp(a)
            g = s * (_CHUNKS * 128) + j * 128 + t * 16 + iota16
            exv[j, pl.ds(t * 16, 16)] = jnp.where(g < E, ex, 0.0)
        pltpu.sync_copy(exv.at[j], den_sh.at[dstv.at[j]], add=True)
        return 0
    lax.fori_loop(0, _CHUNKS, _pa, 0)
    plsc.subcore_barrier()

    pltpu.sync_copy(exv, ex_hbm.at[c, s])
    pltpu.sync_copy(den_sh.at[pl.ds(s * 640, 640)],
                    den_hbm.at[c, pl.ds(s * 640, 640)])


_edge_a = functools.partial(
    pl.kernel,
    out_type=[
        jax.ShapeDtypeStruct((2, 16, _CHUNKS, 128), jnp.float32),
        jax.ShapeDtypeStruct((2, N_PAD), jnp.float32),
    ],
    mesh=_SC_MESH,
    compiler_params=pltpu.CompilerParams(needs_layout_passes=False),
    scratch_types=[
        pltpu.VMEM((N_PAD,), jnp.float32),
        pltpu.VMEM((N_PAD,), jnp.float32),
        pltpu.VMEM((_CHUNKS, 128), jnp.int32),
        pltpu.VMEM((_CHUNKS, 128), jnp.int32),
        pltpu.VMEM((_CHUNKS, 128), jnp.float32),
        pltpu.VMEM((640,), jnp.float32),
        pltpu.VMEM_SHARED((N_PAD,), jnp.float32),
    ],
)(_edge_a_body)


def _edge_c_body(h_hbm, ex_hbm, den_hbm, src_hbm, dst_hbm, out_hbm,
                 denv, srcc, dstc, exc, coefbuf, rows, out_sh, semG, semS):
    c = lax.axis_index("c")
    s = lax.axis_index("s")
    zero16 = jnp.zeros((16,), jnp.float32)

    pltpu.sync_copy(den_hbm.at[c], denv)

    def _zr(j, _):
        for t in range(8):
            rows[0, j, pl.ds(t * 16, 16)] = zero16
        return 0
    lax.fori_loop(0, 128, _zr, 0)
    for k in range(5):
        pltpu.sync_copy(rows.at[0], out_sh.at[pl.ds(s * 640 + k * 128, 128)])
    plsc.subcore_barrier()

    def _prefetch(j, b):
        pltpu.sync_copy(src_hbm.at[c, s, j], srcc.at[b])
        pltpu.sync_copy(dst_hbm.at[c, s, j], dstc.at[b])
        pltpu.sync_copy(ex_hbm.at[c, s, j], exc.at[b])

    def _fire_gather(b):
        pltpu.async_copy(h_hbm.at[srcc.at[b]], rows.at[b], semG)

    def _wait_gather(b):
        pltpu.make_async_copy(h_hbm.at[srcc.at[b]], rows.at[b], semG).wait()

    def _drain_scatter(b):
        pltpu.make_async_copy(h_hbm.at[srcc.at[b]], rows.at[b], semS).wait()

    _prefetch(0, 0)
    _fire_gather(0)

    def _pc(g, _):
        for b in range(2):
            j = 2 * g + b
            nxt = 1 - b

            @pl.when(j >= 1)
            def _():
                _drain_scatter(nxt)

            @pl.when(j <= _CHUNKS - 2)
            def _():
                _prefetch(j + 1, nxt)
                _fire_gather(nxt)

            _wait_gather(b)

            for t in range(8):
                dv = dstc[b, pl.ds(t * 16, 16)]
                den = plsc.load_gather(denv, [dv])
                ex = exc[b, pl.ds(t * 16, 16)]
                coefbuf[pl.ds(t * 16, 16)] = jnp.where(den > 0.0,
                                                       ex / den, 0.0)

            for bb in range(8):
                cv = coefbuf[pl.ds(bb * 16, 16)]
                for q in range(16):
                    cs = cv[q]
                    jj = bb * 16 + q
                    for t in range(8):
                        rows[b, jj, pl.ds(t * 16, 16)] = (
                            rows[b, jj, pl.ds(t * 16, 16)] * cs)
            pltpu.async_copy(rows.at[b], out_sh.at[dstc.at[b]], semS,
                             add=True)
        return 0
    lax.fori_loop(0, _CHUNKS // 2, _pc, 0)
    _drain_scatter(1)
    plsc.subcore_barrier()

    for k in range(5):
        pltpu.sync_copy(out_sh.at[pl.ds(s * 640 + k * 128, 128)],
                        out_hbm.at[c, pl.ds(s * 640 + k * 128, 128)])


_edge_c = functools.partial(
    pl.kernel,
    out_type=jax.ShapeDtypeStruct((2, N_PAD, D), jnp.float32),
    mesh=_SC_MESH,
    compiler_params=pltpu.CompilerParams(needs_layout_passes=False),
    scratch_types=[
        pltpu.VMEM((N_PAD,), jnp.float32),
        pltpu.VMEM((2, 128), jnp.int32),
        pltpu.VMEM((2, 128), jnp.int32),
        pltpu.VMEM((2, 128), jnp.float32),
        pltpu.VMEM((128,), jnp.float32),
        pltpu.VMEM((2, 128, 128), jnp.float32),
        pltpu.VMEM_SHARED((N_PAD, D), jnp.float32),
        pltpu.SemaphoreType.DMA,
        pltpu.SemaphoreType.DMA,
    ],
)(_edge_c_body)


def _prep_edges(ei_next, ei_near):
    def pr(v):
        return jnp.pad(v, (0, E_PAD - E)).reshape(16, _CHUNKS, 128)
    src = jnp.stack([pr(ei_next[0]), pr(ei_near[0])])
    dst = jnp.stack([pr(ei_next[1]), pr(ei_near[1])])
    return src, dst


def kernel(x_traj_point, edge_index_next, edge_index_near, batch_traj_point,
           params):
    srcE, dstE = _prep_edges(edge_index_next, edge_index_near)
    h = x_traj_point
    for l, lp in enumerate(params["layers"]):
        attcat = jnp.stack(
            [lp["att_src"][0], lp["att_dst"][0],
             lp["att_src"][1], lp["att_dst"][1]], axis=1)
        attcat = jnp.pad(attcat, ((0, 0), (0, 4)))
        hp, A = _proj_att(h, lp["proj_W"], lp["proj_b"].reshape(1, D), attcat)
        Ap = jnp.pad(A, ((0, N_PAD - N_NODES), (0, 0)))
        asA = jnp.stack([Ap[:, 0], Ap[:, 2]])
        adA = jnp.stack([Ap[:, 1], Ap[:, 3]])
        ex, den = _edge_a(asA, adA, srcE, dstE)
        p = _edge_c(hp, ex, den, srcE, dstE)
        o, ssum = _combine_a(p, lp["k_W"], lp["k_b"].reshape(1, D))
        h = _combine_b(o, ssum, lp["q"].reshape(1, D),
                       apply_relu=(l < len(params["layers"]) - 1))
    batch3 = batch_traj_point.reshape(_GRID, 1, _BM)
    return _pool_head(h, batch3, params["lin_W"],
                      params["lin_b"].reshape(1, NUM_CLASSES))


# speedup vs baseline: 1.0420x; 1.0420x over previous
import functools

import jax
import jax.numpy as jnp
from jax import lax
from jax.experimental import pallas as pl
from jax.experimental.pallas import tpu as pltpu
from jax.experimental.pallas import tpu_sc as plsc

N_NODES = 10000
D = 128
NUM_CLASSES = 10
NUM_GRAPHS = 64
NEG_SLOPE = 0.2
E = 160000

_BM = 1000
_GRID = N_NODES // _BM

_INTERPRET = False


def _proj_att_body(x_ref, w_ref, b_ref, att_ref, h_ref, a_ref):
    h = jnp.dot(x_ref[...], w_ref[...], preferred_element_type=jnp.float32)
    h = h + b_ref[...]
    h_ref[...] = h
    a_ref[...] = jnp.dot(h, att_ref[...], preferred_element_type=jnp.float32)


def _proj_att(x, W, b, attcat):
    return pl.pallas_call(
        _proj_att_body,
        grid=(_GRID,),
        in_specs=[
            pl.BlockSpec((_BM, D), lambda i: (i, 0)),
            pl.BlockSpec((D, D), lambda i: (0, 0)),
            pl.BlockSpec((1, D), lambda i: (0, 0)),
            pl.BlockSpec((D, 8), lambda i: (0, 0)),
        ],
        out_specs=[
            pl.BlockSpec((_BM, D), lambda i: (i, 0)),
            pl.BlockSpec((_BM, 8), lambda i: (i, 0)),
        ],
        out_shape=[
            jax.ShapeDtypeStruct((N_NODES, D), jnp.float32),
            jax.ShapeDtypeStruct((N_NODES, 8), jnp.float32),
        ],
        interpret=_INTERPRET,
    )(x, W, b, attcat)


def _combine_a_body(p_ref, kw_ref, kb_ref, o_ref, ssum_ref):
    i = pl.program_id(0)
    o0 = jnp.maximum(p_ref[0, :, :], 0.0)
    o1 = jnp.maximum(p_ref[1, :, :], 0.0)
    o_ref[0, :, :] = o0
    o_ref[1, :, :] = o1
    k0 = jnp.tanh(jnp.dot(o0, kw_ref[...], preferred_element_type=jnp.float32)
                  + kb_ref[...])
    k1 = jnp.tanh(jnp.dot(o1, kw_ref[...], preferred_element_type=jnp.float32)
                  + kb_ref[...])
    s = jnp.stack([jnp.sum(k0, axis=0), jnp.sum(k1, axis=0)], axis=0)

    @pl.when(i == 0)
    def _():
        ssum_ref[...] = s

    @pl.when(i > 0)
    def _():
        ssum_ref[...] = ssum_ref[...] + s


def _combine_a(p, kW, kb):
    return pl.pallas_call(
        _combine_a_body,
        grid=(_GRID,),
        in_specs=[
            pl.BlockSpec((2, _BM, D), lambda i: (0, i, 0)),
            pl.BlockSpec((D, D), lambda i: (0, 0)),
            pl.BlockSpec((1, D), lambda i: (0, 0)),
        ],
        out_specs=[
            pl.BlockSpec((2, _BM, D), lambda i: (0, i, 0)),
            pl.BlockSpec((2, D), lambda i: (0, 0)),
        ],
        out_shape=[
            jax.ShapeDtypeStruct((2, N_NODES, D), jnp.float32),
            jax.ShapeDtypeStruct((2, D), jnp.float32),
        ],
        interpret=_INTERPRET,
    )(p, kW, kb)


def _combine_b_body(apply_relu, o_ref, ssum_ref, q_ref, h_ref):
    t = q_ref[...] * ssum_ref[...] * (1.0 / N_NODES)
    score = jnp.sum(t, axis=1, keepdims=True)
    m = jnp.max(score, axis=0, keepdims=True)
    e = jnp.exp(score - m)
    a = e / jnp.sum(e, axis=0, keepdims=True)
    out = o_ref[0, :, :] * a[0:1, 0:1] + o_ref[1, :, :] * a[1:2, 0:1]
    if apply_relu:
        out = jnp.maximum(out, 0.0)
    h_ref[...] = out


def _combine_b(o, ssum, q, apply_relu):
    return pl.pallas_call(
        functools.partial(_combine_b_body, apply_relu),
        grid=(_GRID,),
        in_specs=[
            pl.BlockSpec((2, _BM, D), lambda i: (0, i, 0)),
            pl.BlockSpec((2, D), lambda i: (0, 0)),
            pl.BlockSpec((1, D), lambda i: (0, 0)),
        ],
        out_specs=pl.BlockSpec((_BM, D), lambda i: (i, 0)),
        out_shape=jax.ShapeDtypeStruct((N_NODES, D), jnp.float32),
        interpret=_INTERPRET,
    )(o, ssum, q)


def _pool_body(h_ref, b_ref, lw_ref, lb_ref, out_ref, psum_ref, pcnt_ref):
    i = pl.program_id(0)
    br = b_ref[0]
    rows = lax.broadcasted_iota(jnp.int32, (NUM_GRAPHS, _BM), 0)
    PT = (rows == br).astype(jnp.float32)
    ps = lax.dot_general(PT, h_ref[...], (((1,), (0,)), ((), ())),
                         preferred_element_type=jnp.float32)
    ones = jnp.ones((_BM, 8), jnp.float32)
    pc = lax.dot_general(PT, ones, (((1,), (0,)), ((), ())),
                         preferred_element_type=jnp.float32)

    @pl.when(i == 0)
    def _():
        psum_ref[...] = ps
        pcnt_ref[...] = pc

    @pl.when(i > 0)
    def _():
        psum_ref[...] = psum_ref[...] + ps
        pcnt_ref[...] = pcnt_ref[...] + pc

    @pl.when(i == _GRID - 1)
    def _():
        pooled = psum_ref[...] / jnp.maximum(pcnt_ref[:, 0:1], 1.0)
        out_ref[...] = jnp.dot(pooled, lw_ref[...],
                               preferred_element_type=jnp.float32) + lb_ref[...]


def _pool_head(h, batch3, lin_W, lin_b):
    return pl.pallas_call(
        _pool_body,
        grid=(_GRID,),
        in_specs=[
            pl.BlockSpec((_BM, D), lambda i: (i, 0)),
            pl.BlockSpec((1, 1, _BM), lambda i: (i, 0, 0)),
            pl.BlockSpec((D, NUM_CLASSES), lambda i: (0, 0)),
            pl.BlockSpec((1, NUM_CLASSES), lambda i: (0, 0)),
        ],
        out_specs=pl.BlockSpec((NUM_GRAPHS, NUM_CLASSES), lambda i: (0, 0)),
        out_shape=jax.ShapeDtypeStruct((NUM_GRAPHS, NUM_CLASSES), jnp.float32),
        scratch_shapes=[
            pltpu.VMEM((NUM_GRAPHS, D), jnp.float32),
            pltpu.VMEM((NUM_GRAPHS, 8), jnp.float32),
        ],
        interpret=_INTERPRET,
    )(h, batch3, lin_W, lin_b)


E_PAD = 163840
N_PAD = 10240
_CHUNKS = 80

_SC_MESH = plsc.VectorSubcoreMesh(core_axis_name="c", subcore_axis_name="s",
                                  num_cores=2, num_subcores=16)


def _edge_a_body(as_hbm, ad_hbm, src_hbm, dst_hbm, ex_hbm, den_hbm,
                 asv, adv, srcv, dstv, exv, zbuf, den_sh):
    c = lax.axis_index("c")
    s = lax.axis_index("s")
    zero16 = jnp.zeros((16,), jnp.float32)
    iota16 = lax.iota(jnp.int32, 16)

    pltpu.sync_copy(as_hbm.at[c], asv)
    pltpu.sync_copy(ad_hbm.at[c], adv)
    pltpu.sync_copy(src_hbm.at[c, s], srcv)
    pltpu.sync_copy(dst_hbm.at[c, s], dstv)

    for t in range(640 // 16):
        zbuf[pl.ds(t * 16, 16)] = zero16
    pltpu.sync_copy(zbuf, den_sh.at[pl.ds(s * 640, 640)])
    plsc.subcore_barrier()

    def _pa(j, _):
        for t in range(8):
            sv = srcv[j, pl.ds(t * 16, 16)]
            dv = dstv[j, pl.ds(t * 16, 16)]
            a = plsc.load_gather(asv, [sv]) + plsc.load_gather(adv, [dv])
            a = jnp.where(a > 0, a, NEG_SLOPE * a)
            ex = jnp.exp(a)
            g = s * (_CHUNKS * 128) + j * 128 + t * 16 + iota16
            exv[j, pl.ds(t * 16, 16)] = jnp.where(g < E, ex, 0.0)
        pltpu.sync_copy(exv.at[j], den_sh.at[dstv.at[j]], add=True)
        return 0
    lax.fori_loop(0, _CHUNKS, _pa, 0)
    plsc.subcore_barrier()

    pltpu.sync_copy(exv, ex_hbm.at[c, s])
    pltpu.sync_copy(den_sh.at[pl.ds(s * 640, 640)],
                    den_hbm.at[c, pl.ds(s * 640, 640)])


_edge_a = functools.partial(
    pl.kernel,
    out_type=[
        jax.ShapeDtypeStruct((2, 16, _CHUNKS, 128), jnp.float32),
        jax.ShapeDtypeStruct((2, N_PAD), jnp.float32),
    ],
    mesh=_SC_MESH,
    compiler_params=pltpu.CompilerParams(needs_layout_passes=False),
    scratch_types=[
        pltpu.VMEM((N_PAD,), jnp.float32),
        pltpu.VMEM((N_PAD,), jnp.float32),
        pltpu.VMEM((_CHUNKS, 128), jnp.int32),
        pltpu.VMEM((_CHUNKS, 128), jnp.int32),
        pltpu.VMEM((_CHUNKS, 128), jnp.float32),
        pltpu.VMEM((640,), jnp.float32),
        pltpu.VMEM_SHARED((N_PAD,), jnp.float32),
    ],
)(_edge_a_body)


def _edge_c_body(h_hbm, ex_hbm, den_hbm, src_hbm, dst_hbm, out_hbm,
                 denv, srcc, dstc, exc, coefbuf, rows, out_sh, semG, semS):
    c = lax.axis_index("c")
    s = lax.axis_index("s")
    zero16 = jnp.zeros((16,), jnp.float32)

    pltpu.sync_copy(den_hbm.at[c], denv)

    def _zr(j, _):
        for t in range(8):
            rows[0, j, pl.ds(t * 16, 16)] = zero16
        return 0
    lax.fori_loop(0, 128, _zr, 0)
    for k in range(5):
        pltpu.sync_copy(rows.at[0], out_sh.at[pl.ds(s * 640 + k * 128, 128)])
    plsc.subcore_barrier()

    def _prefetch(j, b):
        pltpu.sync_copy(src_hbm.at[c, s, j], srcc.at[b])
        pltpu.sync_copy(dst_hbm.at[c, s, j], dstc.at[b])
        pltpu.sync_copy(ex_hbm.at[c, s, j], exc.at[b])

    def _fire_gather(b):
        pltpu.async_copy(h_hbm.at[srcc.at[b]], rows.at[b], semG)

    def _wait_gather(b):
        pltpu.make_async_copy(h_hbm.at[srcc.at[b]], rows.at[b], semG).wait()

    def _drain_scatter(b):
        pltpu.make_async_copy(h_hbm.at[srcc.at[b]], rows.at[b], semS).wait()

    _prefetch(0, 0)
    _fire_gather(0)

    def _pc(g, _):
        for b in range(2):
            j = 2 * g + b
            nxt = 1 - b

            @pl.when(j >= 1)
            def _():
                _drain_scatter(nxt)

            @pl.when(j <= _CHUNKS - 2)
            def _():
                _prefetch(j + 1, nxt)
                _fire_gather(nxt)

            _wait_gather(b)

            for t in range(8):
                dv = dstc[b, pl.ds(t * 16, 16)]
                den = plsc.load_gather(denv, [dv])
                ex = exc[b, pl.ds(t * 16, 16)]
                coefbuf[pl.ds(t * 16, 16)] = jnp.where(den > 0.0,
                                                       ex / den, 0.0)

            def _mrow(bb, _):
                cv = coefbuf[pl.ds(bb * 16, 16)]
                for q in range(16):
                    cs = cv[q]
                    jj = bb * 16 + q
                    for t in range(8):
                        rows[b, jj, pl.ds(t * 16, 16)] = (
                            rows[b, jj, pl.ds(t * 16, 16)] * cs)
                return 0
            lax.fori_loop(0, 8, _mrow, 0)
            pltpu.async_copy(rows.at[b], out_sh.at[dstc.at[b]], semS,
                             add=True)
        return 0
    lax.fori_loop(0, _CHUNKS // 2, _pc, 0)
    _drain_scatter(1)
    plsc.subcore_barrier()

    for k in range(5):
        pltpu.sync_copy(out_sh.at[pl.ds(s * 640 + k * 128, 128)],
                        out_hbm.at[c, pl.ds(s * 640 + k * 128, 128)])


_edge_c = functools.partial(
    pl.kernel,
    out_type=jax.ShapeDtypeStruct((2, N_PAD, D), jnp.float32),
    mesh=_SC_MESH,
    compiler_params=pltpu.CompilerParams(needs_layout_passes=False),
    scratch_types=[
        pltpu.VMEM((N_PAD,), jnp.float32),
        pltpu.VMEM((2, 128), jnp.int32),
        pltpu.VMEM((2, 128), jnp.int32),
        pltpu.VMEM((2, 128), jnp.float32),
        pltpu.VMEM((128,), jnp.float32),
        pltpu.VMEM((2, 128, 128), jnp.float32),
        pltpu.VMEM_SHARED((N_PAD, D), jnp.float32),
        pltpu.SemaphoreType.DMA,
        pltpu.SemaphoreType.DMA,
    ],
)(_edge_c_body)


def _prep_edges(ei_next, ei_near):
    def pr(v):
        return jnp.pad(v, (0, E_PAD - E)).reshape(16, _CHUNKS, 128)
    src = jnp.stack([pr(ei_next[0]), pr(ei_near[0])])
    dst = jnp.stack([pr(ei_next[1]), pr(ei_near[1])])
    return src, dst


def kernel(x_traj_point, edge_index_next, edge_index_near, batch_traj_point,
           params):
    srcE, dstE = _prep_edges(edge_index_next, edge_index_near)
    h = x_traj_point
    for l, lp in enumerate(params["layers"]):
        attcat = jnp.stack(
            [lp["att_src"][0], lp["att_dst"][0],
             lp["att_src"][1], lp["att_dst"][1]], axis=1)
        attcat = jnp.pad(attcat, ((0, 0), (0, 4)))
        hp, A = _proj_att(h, lp["proj_W"], lp["proj_b"].reshape(1, D), attcat)
        Ap = jnp.pad(A, ((0, N_PAD - N_NODES), (0, 0)))
        asA = jnp.stack([Ap[:, 0], Ap[:, 2]])
        adA = jnp.stack([Ap[:, 1], Ap[:, 3]])
        ex, den = _edge_a(asA, adA, srcE, dstE)
        p = _edge_c(hp, ex, den, srcE, dstE)
        o, ssum = _combine_a(p, lp["k_W"], lp["k_b"].reshape(1, D))
        h = _combine_b(o, ssum, lp["q"].reshape(1, D),
                       apply_relu=(l < len(params["layers"]) - 1))
    batch3 = batch_traj_point.reshape(_GRID, 1, _BM)
    return _pool_head(h, batch3, params["lin_W"],
                      params["lin_b"].reshape(1, NUM_CLASSES))
